# initial kernel scaffold (unmeasured)
import functools

import jax
import jax.numpy as jnp
from jax import lax
from jax.experimental import pallas as pl
from jax.experimental.pallas import tpu as pltpu

N_DEV = 32
LOG2_N = 5
N_HEADS = 8
DH = 128
SQ = 256
D_MODEL = 1024
SCALE = 0.08838834764831843


def kernel(x, Wq, Wo, Wk, Wv):
    x2 = x.reshape(SQ, D_MODEL)

    def body(x_ref, wq_ref, wo_ref, wk_ref, wv_ref, out_ref,
             acc_ref, recv_ref, send_sems, recv_sems):
        my = lax.axis_index("i")

        q = jnp.dot(x_ref[:], wq_ref[:], preferred_element_type=jnp.float32)
        k = jnp.dot(x_ref[:], wk_ref[:], preferred_element_type=jnp.float32)
        v = jnp.dot(x_ref[:], wv_ref[:], preferred_element_type=jnp.float32)

        outs = []
        for h in range(N_HEADS):
            qh = q[:, h * DH:(h + 1) * DH]
            kh = k[:, h * DH:(h + 1) * DH]
            vh = v[:, h * DH:(h + 1) * DH]
            s = lax.dot_general(
                qh, kh, (((1,), (1,)), ((), ())),
                preferred_element_type=jnp.float32,
            ) * SCALE
            m = jnp.max(s, axis=1, keepdims=True)
            p = jnp.exp(s - m)
            l = jnp.sum(p, axis=1, keepdims=True)
            outs.append(
                jnp.dot(p, vh, preferred_element_type=jnp.float32) / l
            )
        attn = jnp.concatenate(outs, axis=1)
        acc_ref[:] = jnp.dot(attn, wo_ref[:],
                             preferred_element_type=jnp.float32)

        for r in range(LOG2_N):
            partner = my ^ (1 << r)
            rdma = pltpu.make_async_remote_copy(
                src_ref=acc_ref,
                dst_ref=recv_ref.at[r],
                send_sem=send_sems.at[r],
                recv_sem=recv_sems.at[r],
                device_id=(partner,),
                device_id_type=pl.DeviceIdType.MESH,
            )
            rdma.start()
            rdma.wait()
            acc_ref[:] = acc_ref[:] + recv_ref[r]

        out_ref[:] = acc_ref[:]

    out = pl.pallas_call(
        body,
        out_shape=jax.ShapeDtypeStruct((SQ, D_MODEL), jnp.float32),
        in_specs=[pl.BlockSpec(memory_space=pltpu.VMEM)] * 5,
        out_specs=pl.BlockSpec(memory_space=pltpu.VMEM),
        scratch_shapes=[
            pltpu.VMEM((SQ, D_MODEL), jnp.float32),
            pltpu.VMEM((LOG2_N, SQ, D_MODEL), jnp.float32),
            pltpu.SemaphoreType.DMA((LOG2_N,)),
            pltpu.SemaphoreType.DMA((LOG2_N,)),
        ],
        compiler_params=pltpu.CompilerParams(collective_id=0),
    )(x2, Wq, Wo, Wk, Wv)
    return out.reshape(1, SQ, D_MODEL)


# baseline (device time: 116889 ns/iter reference)
import functools

import jax
import jax.numpy as jnp
from jax import lax
from jax.experimental import pallas as pl
from jax.experimental.pallas import tpu as pltpu

N_DEV = 32
LOG2_N = 5
N_HEADS = 8
DH = 128
SQ = 256
D_MODEL = 1024
SCALE = 0.08838834764831843


def kernel(x, Wq, Wo, Wk, Wv):
    x2 = x.reshape(SQ, D_MODEL)

    def body(x_ref, wq_ref, wo_ref, wk_ref, wv_ref, out_ref,
             acc_ref, recv_ref, send_sems, recv_sems):
        my = lax.axis_index("i")

        q = jnp.dot(x_ref[:], wq_ref[:], preferred_element_type=jnp.float32)
        k = jnp.dot(x_ref[:], wk_ref[:], preferred_element_type=jnp.float32)
        v = jnp.dot(x_ref[:], wv_ref[:], preferred_element_type=jnp.float32)

        outs = []
        for h in range(N_HEADS):
            qh = q[:, h * DH:(h + 1) * DH]
            kh = k[:, h * DH:(h + 1) * DH]
            vh = v[:, h * DH:(h + 1) * DH]
            s = lax.dot_general(
                qh, kh, (((1,), (1,)), ((), ())),
                preferred_element_type=jnp.float32,
            ) * SCALE
            m = jnp.max(s, axis=1, keepdims=True)
            p = jnp.exp(s - m)
            l = jnp.sum(p, axis=1, keepdims=True)
            outs.append(
                jnp.dot(p, vh, preferred_element_type=jnp.float32) / l
            )
        attn = jnp.concatenate(outs, axis=1)
        acc_ref[:] = jnp.dot(attn, wo_ref[:],
                             preferred_element_type=jnp.float32)

        for r in range(LOG2_N):
            partner = my ^ (1 << r)
            rdma = pltpu.make_async_remote_copy(
                src_ref=acc_ref,
                dst_ref=recv_ref.at[r],
                send_sem=send_sems.at[r],
                recv_sem=recv_sems.at[r],
                device_id=(partner,),
                device_id_type=pl.DeviceIdType.MESH,
            )
            rdma.start()
            rdma.wait()
            acc_ref[:] = acc_ref[:] + recv_ref[r]

        out_ref[:] = acc_ref[:]

    out = pl.pallas_call(
        body,
        out_shape=jax.ShapeDtypeStruct((SQ, D_MODEL), jnp.float32),
        in_specs=[pl.BlockSpec(memory_space=pltpu.VMEM)] * 5,
        out_specs=pl.BlockSpec(memory_space=pltpu.VMEM),
        scratch_shapes=[
            pltpu.VMEM((SQ, D_MODEL), jnp.float32),
            pltpu.VMEM((LOG2_N, SQ, D_MODEL), jnp.float32),
            pltpu.SemaphoreType.DMA((LOG2_N,)),
            pltpu.SemaphoreType.DMA((LOG2_N,)),
        ],
    )(x2, Wq, Wo, Wk, Wv)
    return out.reshape(1, SQ, D_MODEL)


# device time: 66374 ns/iter; 1.7611x vs baseline; 1.7611x over previous
import functools

import jax
import jax.numpy as jnp
from jax import lax
from jax.experimental import pallas as pl
from jax.experimental.pallas import tpu as pltpu

N_DEV = 32
LOG2_N = 5
N_HEADS = 8
DH = 128
SQ = 256
D_MODEL = 1024
SCALE = 0.08838834764831843


def kernel(x, Wq, Wo, Wk, Wv):
    x2 = x.reshape(SQ, D_MODEL)

    RS_BITS = (0, 3, 1, 2, 4)

    def body(x_ref, wq_ref, wo_ref, wk_ref, wv_ref, out_ref,
             acc_ref, r0, r1, r2, r3, r4,
             rs_send_sems, rs_recv_sems, ag_send_sems, ag_recv_sems):
        my = lax.axis_index("i")
        recvs = [r0, r1, r2, r3, r4]

        q = jnp.dot(x_ref[:], wq_ref[:], preferred_element_type=jnp.float32)
        k = jnp.dot(x_ref[:], wk_ref[:], preferred_element_type=jnp.float32)
        v = jnp.dot(x_ref[:], wv_ref[:], preferred_element_type=jnp.float32)

        outs = []
        for h in range(N_HEADS):
            qh = q[:, h * DH:(h + 1) * DH]
            kh = k[:, h * DH:(h + 1) * DH]
            vh = v[:, h * DH:(h + 1) * DH]
            s = lax.dot_general(
                qh, kh, (((1,), (1,)), ((), ())),
                preferred_element_type=jnp.float32,
            ) * SCALE
            m = jnp.max(s, axis=1, keepdims=True)
            p = jnp.exp(s - m)
            l = jnp.sum(p, axis=1, keepdims=True)
            outs.append(
                jnp.dot(p, vh, preferred_element_type=jnp.float32) / l
            )
        attn = jnp.concatenate(outs, axis=1)
        acc_ref[:] = jnp.dot(attn, wo_ref[:],
                             preferred_element_type=jnp.float32)

        cur_off = 0
        size = SQ
        for j, b in enumerate(RS_BITS):
            half = size // 2
            partner = my ^ (1 << b)
            mybit = (my >> b) & 1
            send_off = cur_off + (1 - mybit) * half
            keep_off = cur_off + mybit * half
            rdma = pltpu.make_async_remote_copy(
                src_ref=acc_ref.at[pl.ds(send_off, half), :],
                dst_ref=recvs[j],
                send_sem=rs_send_sems.at[j],
                recv_sem=rs_recv_sems.at[j],
                device_id=(partner,),
                device_id_type=pl.DeviceIdType.MESH,
            )
            rdma.start()
            rdma.wait()
            acc_ref[pl.ds(keep_off, half), :] = (
                acc_ref[pl.ds(keep_off, half), :] + recvs[j][:, :]
            )
            cur_off = keep_off
            size = half

        for j, b in reversed(list(enumerate(RS_BITS))):
            partner = my ^ (1 << b)
            mybit = (my >> b) & 1
            rdma = pltpu.make_async_remote_copy(
                src_ref=acc_ref.at[pl.ds(cur_off, size), :],
                dst_ref=acc_ref.at[pl.ds(cur_off, size), :],
                send_sem=ag_send_sems.at[j],
                recv_sem=ag_recv_sems.at[j],
                device_id=(partner,),
                device_id_type=pl.DeviceIdType.MESH,
            )
            rdma.start()
            rdma.wait()
            cur_off = cur_off - mybit * size
            size = size * 2

        out_ref[:] = acc_ref[:]

    out = pl.pallas_call(
        body,
        out_shape=jax.ShapeDtypeStruct((SQ, D_MODEL), jnp.float32),
        in_specs=[pl.BlockSpec(memory_space=pltpu.VMEM)] * 5,
        out_specs=pl.BlockSpec(memory_space=pltpu.VMEM),
        scratch_shapes=[
            pltpu.VMEM((SQ, D_MODEL), jnp.float32),
            pltpu.VMEM((128, D_MODEL), jnp.float32),
            pltpu.VMEM((64, D_MODEL), jnp.float32),
            pltpu.VMEM((32, D_MODEL), jnp.float32),
            pltpu.VMEM((16, D_MODEL), jnp.float32),
            pltpu.VMEM((8, D_MODEL), jnp.float32),
            pltpu.SemaphoreType.DMA((LOG2_N,)),
            pltpu.SemaphoreType.DMA((LOG2_N,)),
            pltpu.SemaphoreType.DMA((LOG2_N,)),
            pltpu.SemaphoreType.DMA((LOG2_N,)),
        ],
    )(x2, Wq, Wo, Wk, Wv)
    return out.reshape(1, SQ, D_MODEL)


# device time: 54381 ns/iter; 2.1494x vs baseline; 1.2205x over previous
import jax
import jax.numpy as jnp
from jax import lax
from jax.experimental import pallas as pl
from jax.experimental.pallas import tpu as pltpu

N_DEV = 32
LOG2_N = 5
N_HEADS = 8
DH = 128
SQ = 256
D_MODEL = 1024
SCALE = 0.08838834764831843

RS_BITS = (0, 3, 1, 2, 4)


def kernel(x, Wq, Wo, Wk, Wv):
    xb = x.reshape(SQ, D_MODEL).astype(jnp.bfloat16)
    wqb = Wq.astype(jnp.bfloat16)
    wkb = Wk.astype(jnp.bfloat16)
    wvb = Wv.astype(jnp.bfloat16)
    wob = Wo.astype(jnp.bfloat16)

    def body(x_ref, wq_ref, wo_ref, wk_ref, wv_ref, out_ref,
             acc_ref, ag_ref, send_ref, r0, r1, r2, r3, r4,
             rs_send_sems, rs_recv_sems, ag_send_sems, ag_recv_sems):
        my = lax.axis_index("i")
        recvs = [r0, r1, r2, r3, r4]

        q = jnp.dot(x_ref[:], wq_ref[:],
                    preferred_element_type=jnp.float32).astype(jnp.bfloat16)
        k = jnp.dot(x_ref[:], wk_ref[:],
                    preferred_element_type=jnp.float32).astype(jnp.bfloat16)
        v = jnp.dot(x_ref[:], wv_ref[:],
                    preferred_element_type=jnp.float32).astype(jnp.bfloat16)

        outs = []
        for h in range(N_HEADS):
            qh = q[:, h * DH:(h + 1) * DH]
            kh = k[:, h * DH:(h + 1) * DH]
            vh = v[:, h * DH:(h + 1) * DH]
            s = lax.dot_general(
                qh, kh, (((1,), (1,)), ((), ())),
                preferred_element_type=jnp.float32,
            ) * SCALE
            m = jnp.max(s, axis=1, keepdims=True)
            p = jnp.exp(s - m)
            l = jnp.sum(p, axis=1, keepdims=True)
            ph = p.astype(jnp.bfloat16)
            outs.append(
                jnp.dot(ph, vh, preferred_element_type=jnp.float32) / l
            )
        attn = jnp.concatenate(outs, axis=1).astype(jnp.bfloat16)
        acc_ref[:] = jnp.dot(attn, wo_ref[:],
                             preferred_element_type=jnp.float32)

        cur_off = 0
        size = SQ
        for j, b in enumerate(RS_BITS):
            half = size // 2
            partner = my ^ (1 << b)
            mybit = (my >> b) & 1
            send_off = cur_off + (1 - mybit) * half
            keep_off = cur_off + mybit * half
            send_ref[pl.ds(0, half), :] = (
                acc_ref[pl.ds(send_off, half), :].astype(jnp.bfloat16)
            )
            rdma = pltpu.make_async_remote_copy(
                src_ref=send_ref.at[pl.ds(0, half), :],
                dst_ref=recvs[j],
                send_sem=rs_send_sems.at[j],
                recv_sem=rs_recv_sems.at[j],
                device_id=(partner,),
                device_id_type=pl.DeviceIdType.MESH,
            )
            rdma.start()
            rdma.wait()
            acc_ref[pl.ds(keep_off, half), :] = (
                acc_ref[pl.ds(keep_off, half), :]
                + recvs[j][:, :].astype(jnp.float32)
            )
            cur_off = keep_off
            size = half

        ag_ref[pl.ds(cur_off, size), :] = (
            acc_ref[pl.ds(cur_off, size), :].astype(jnp.bfloat16)
        )
        for j, b in reversed(list(enumerate(RS_BITS))):
            partner = my ^ (1 << b)
            mybit = (my >> b) & 1
            rdma = pltpu.make_async_remote_copy(
                src_ref=ag_ref.at[pl.ds(cur_off, size), :],
                dst_ref=ag_ref.at[pl.ds(cur_off, size), :],
                send_sem=ag_send_sems.at[j],
                recv_sem=ag_recv_sems.at[j],
                device_id=(partner,),
                device_id_type=pl.DeviceIdType.MESH,
            )
            rdma.start()
            rdma.wait()
            cur_off = cur_off - mybit * size
            size = size * 2

        out_ref[:] = ag_ref[:].astype(jnp.float32)

    out = pl.pallas_call(
        body,
        out_shape=jax.ShapeDtypeStruct((SQ, D_MODEL), jnp.float32),
        in_specs=[pl.BlockSpec(memory_space=pltpu.VMEM)] * 5,
        out_specs=pl.BlockSpec(memory_space=pltpu.VMEM),
        scratch_shapes=[
            pltpu.VMEM((SQ, D_MODEL), jnp.float32),
            pltpu.VMEM((SQ, D_MODEL), jnp.bfloat16),
            pltpu.VMEM((128, D_MODEL), jnp.bfloat16),
            pltpu.VMEM((128, D_MODEL), jnp.bfloat16),
            pltpu.VMEM((64, D_MODEL), jnp.bfloat16),
            pltpu.VMEM((32, D_MODEL), jnp.bfloat16),
            pltpu.VMEM((16, D_MODEL), jnp.bfloat16),
            pltpu.VMEM((8, D_MODEL), jnp.bfloat16),
            pltpu.SemaphoreType.DMA((LOG2_N,)),
            pltpu.SemaphoreType.DMA((LOG2_N,)),
            pltpu.SemaphoreType.DMA((LOG2_N,)),
            pltpu.SemaphoreType.DMA((LOG2_N,)),
        ],
    )(xb, wqb, wob, wkb, wvb)
    return out.reshape(1, SQ, D_MODEL)


# device time: 43452 ns/iter; 2.6901x vs baseline; 1.2515x over previous
import jax
import jax.numpy as jnp
from jax import lax
from jax.experimental import pallas as pl
from jax.experimental.pallas import tpu as pltpu

N_DEV = 32
LOG2_N = 5
N_HEADS = 8
DH = 128
SQ = 256
D_MODEL = 1024
SCALE = 0.08838834764831843

CHUNK = SQ // N_DEV


def kernel(x, Wq, Wo, Wk, Wv):
    xb = x.reshape(SQ, D_MODEL).astype(jnp.bfloat16)
    wqb = Wq.astype(jnp.bfloat16)
    wkb = Wk.astype(jnp.bfloat16)
    wvb = Wv.astype(jnp.bfloat16)
    wob = Wo.astype(jnp.bfloat16)

    def body(x_ref, wq_ref, wo_ref, wk_ref, wv_ref, out_ref,
             acc_ref, ag_ref, stage_ref, rs_recv,
             rs_send_sems, rs_recv_sems, ag_send_sems, ag_recv_sems):
        my = lax.axis_index("i")

        q = jnp.dot(x_ref[:], wq_ref[:],
                    preferred_element_type=jnp.float32).astype(jnp.bfloat16)
        k = jnp.dot(x_ref[:], wk_ref[:],
                    preferred_element_type=jnp.float32).astype(jnp.bfloat16)
        v = jnp.dot(x_ref[:], wv_ref[:],
                    preferred_element_type=jnp.float32).astype(jnp.bfloat16)

        outs = []
        for h in range(N_HEADS):
            qh = q[:, h * DH:(h + 1) * DH]
            kh = k[:, h * DH:(h + 1) * DH]
            vh = v[:, h * DH:(h + 1) * DH]
            s = lax.dot_general(
                qh, kh, (((1,), (1,)), ((), ())),
                preferred_element_type=jnp.float32,
            ) * SCALE
            m = jnp.max(s, axis=1, keepdims=True)
            p = jnp.exp(s - m)
            l = jnp.sum(p, axis=1, keepdims=True)
            ph = p.astype(jnp.bfloat16)
            outs.append(
                jnp.dot(ph, vh, preferred_element_type=jnp.float32) / l
            )
        attn = jnp.concatenate(outs, axis=1).astype(jnp.bfloat16)
        acc_ref[:] = jnp.dot(attn, wo_ref[:],
                             preferred_element_type=jnp.float32)

        stage_ref[:] = acc_ref[:].astype(jnp.bfloat16)
        rs_sends = []
        for j in range(N_DEV - 1):
            d = lax.rem(my + 1 + j, N_DEV)
            rdma = pltpu.make_async_remote_copy(
                src_ref=stage_ref.at[pl.ds(CHUNK * d, CHUNK), :],
                dst_ref=rs_recv.at[30 - j],
                send_sem=rs_send_sems.at[j],
                recv_sem=rs_recv_sems.at[30 - j],
                device_id=(d,),
                device_id_type=pl.DeviceIdType.MESH,
            )
            rdma.start()
            rs_sends.append(rdma)

        my_off = CHUNK * my
        red = acc_ref[pl.ds(my_off, CHUNK), :]
        for s in range(N_DEV - 1):
            recv = pltpu.make_async_remote_copy(
                src_ref=rs_recv.at[s],
                dst_ref=rs_recv.at[s],
                send_sem=rs_send_sems.at[0],
                recv_sem=rs_recv_sems.at[s],
                device_id=(my,),
                device_id_type=pl.DeviceIdType.MESH,
            )
            recv.wait_recv()
            red = red + rs_recv[s].astype(jnp.float32)

        ag_ref[pl.ds(my_off, CHUNK), :] = red.astype(jnp.bfloat16)
        ag_sends = []
        for j in range(N_DEV - 1):
            d = lax.rem(my + 1 + j, N_DEV)
            rdma = pltpu.make_async_remote_copy(
                src_ref=ag_ref.at[pl.ds(my_off, CHUNK), :],
                dst_ref=ag_ref.at[pl.ds(my_off, CHUNK), :],
                send_sem=ag_send_sems.at[j],
                recv_sem=ag_recv_sems.at[30 - j],
                device_id=(d,),
                device_id_type=pl.DeviceIdType.MESH,
            )
            rdma.start()
            ag_sends.append(rdma)

        for s in range(N_DEV - 1):
            src = lax.rem(my + 1 + s, N_DEV)
            recv = pltpu.make_async_remote_copy(
                src_ref=ag_ref.at[pl.ds(CHUNK * src, CHUNK), :],
                dst_ref=ag_ref.at[pl.ds(CHUNK * src, CHUNK), :],
                send_sem=ag_send_sems.at[0],
                recv_sem=ag_recv_sems.at[s],
                device_id=(my,),
                device_id_type=pl.DeviceIdType.MESH,
            )
            recv.wait_recv()

        for rdma in rs_sends:
            rdma.wait_send()
        for rdma in ag_sends:
            rdma.wait_send()

        out_ref[:] = ag_ref[:].astype(jnp.float32)

    out = pl.pallas_call(
        body,
        out_shape=jax.ShapeDtypeStruct((SQ, D_MODEL), jnp.float32),
        in_specs=[pl.BlockSpec(memory_space=pltpu.VMEM)] * 5,
        out_specs=pl.BlockSpec(memory_space=pltpu.VMEM),
        scratch_shapes=[
            pltpu.VMEM((SQ, D_MODEL), jnp.float32),
            pltpu.VMEM((SQ, D_MODEL), jnp.bfloat16),
            pltpu.VMEM((SQ, D_MODEL), jnp.bfloat16),
            pltpu.VMEM((N_DEV - 1, CHUNK, D_MODEL), jnp.bfloat16),
            pltpu.SemaphoreType.DMA((N_DEV - 1,)),
            pltpu.SemaphoreType.DMA((N_DEV - 1,)),
            pltpu.SemaphoreType.DMA((N_DEV - 1,)),
            pltpu.SemaphoreType.DMA((N_DEV - 1,)),
        ],
    )(xb, wqb, wob, wkb, wvb)
    return out.reshape(1, SQ, D_MODEL)


# device time: 34865 ns/iter; 3.3526x vs baseline; 1.2463x over previous
import jax
import jax.numpy as jnp
from jax import lax
from jax.experimental import pallas as pl
from jax.experimental.pallas import tpu as pltpu

N_DEV = 32
LOG2_N = 5
N_HEADS = 8
DH = 128
SQ = 256
D_MODEL = 1024
SCALE = 0.08838834764831843

CHUNK = SQ // N_DEV


def kernel(x, Wq, Wo, Wk, Wv):
    xb = x.reshape(SQ, D_MODEL).astype(jnp.bfloat16)
    wqb = Wq.astype(jnp.bfloat16)
    wkb = Wk.astype(jnp.bfloat16)
    wvb = Wv.astype(jnp.bfloat16)
    wob = Wo.astype(jnp.bfloat16)

    def body(x_ref, wq_ref, wo_ref, wk_ref, wv_ref, out_ref,
             acc_ref, ag_ref, rs_recv,
             rs_send_sems, rs_recv_sems, ag_send_sems, ag_recv_sems):
        my = lax.axis_index("i")

        barrier_sem = pltpu.get_barrier_semaphore()
        for j in range(N_DEV - 1):
            d = lax.rem(my + 1 + j, N_DEV)
            pl.semaphore_signal(
                barrier_sem, inc=1,
                device_id=(d,), device_id_type=pl.DeviceIdType.MESH,
            )

        q = jnp.dot(x_ref[:], wq_ref[:],
                    preferred_element_type=jnp.float32).astype(jnp.bfloat16)
        k = jnp.dot(x_ref[:], wk_ref[:],
                    preferred_element_type=jnp.float32).astype(jnp.bfloat16)
        v = jnp.dot(x_ref[:], wv_ref[:],
                    preferred_element_type=jnp.float32).astype(jnp.bfloat16)

        outs = []
        for h in range(N_HEADS):
            qh = q[:, h * DH:(h + 1) * DH]
            kh = k[:, h * DH:(h + 1) * DH]
            vh = v[:, h * DH:(h + 1) * DH]
            s = lax.dot_general(
                qh, kh, (((1,), (1,)), ((), ())),
                preferred_element_type=jnp.float32,
            ) * SCALE
            m = jnp.max(s, axis=1, keepdims=True)
            p = jnp.exp(s - m)
            l = jnp.sum(p, axis=1, keepdims=True)
            ph = p.astype(jnp.bfloat16)
            outs.append(
                jnp.dot(ph, vh, preferred_element_type=jnp.float32) / l
            )
        attn = jnp.concatenate(outs, axis=1).astype(jnp.bfloat16)
        acc_ref[:] = jnp.dot(attn, wo_ref[:],
                             preferred_element_type=jnp.float32
                             ).astype(jnp.bfloat16)

        pl.semaphore_wait(barrier_sem, N_DEV - 1)
        rs_sends = []
        for j in range(N_DEV - 1):
            d = lax.rem(my + 1 + j, N_DEV)
            rdma = pltpu.make_async_remote_copy(
                src_ref=acc_ref.at[pl.ds(CHUNK * d, CHUNK), :],
                dst_ref=rs_recv.at[30 - j],
                send_sem=rs_send_sems.at[j],
                recv_sem=rs_recv_sems.at[30 - j],
                device_id=(d,),
                device_id_type=pl.DeviceIdType.MESH,
            )
            rdma.start()
            rs_sends.append(rdma)

        my_off = CHUNK * my
        for s in range(N_DEV - 1):
            recv = pltpu.make_async_remote_copy(
                src_ref=rs_recv.at[s],
                dst_ref=rs_recv.at[s],
                send_sem=rs_send_sems.at[0],
                recv_sem=rs_recv_sems.at[s],
                device_id=(my,),
                device_id_type=pl.DeviceIdType.MESH,
            )
            recv.wait_recv()
        red = acc_ref[pl.ds(my_off, CHUNK), :].astype(jnp.float32) + jnp.sum(
            rs_recv[:, :, :].astype(jnp.float32), axis=0
        )

        ag_ref[pl.ds(my_off, CHUNK), :] = red.astype(jnp.bfloat16)
        ag_sends = []
        for j in range(N_DEV - 1):
            d = lax.rem(my + 1 + j, N_DEV)
            rdma = pltpu.make_async_remote_copy(
                src_ref=ag_ref.at[pl.ds(my_off, CHUNK), :],
                dst_ref=ag_ref.at[pl.ds(my_off, CHUNK), :],
                send_sem=ag_send_sems.at[j],
                recv_sem=ag_recv_sems.at[30 - j],
                device_id=(d,),
                device_id_type=pl.DeviceIdType.MESH,
            )
            rdma.start()
            ag_sends.append(rdma)

        for s in range(N_DEV - 1):
            src = lax.rem(my + 1 + s, N_DEV)
            recv = pltpu.make_async_remote_copy(
                src_ref=ag_ref.at[pl.ds(CHUNK * src, CHUNK), :],
                dst_ref=ag_ref.at[pl.ds(CHUNK * src, CHUNK), :],
                send_sem=ag_send_sems.at[0],
                recv_sem=ag_recv_sems.at[s],
                device_id=(my,),
                device_id_type=pl.DeviceIdType.MESH,
            )
            recv.wait_recv()

        for rdma in rs_sends:
            rdma.wait_send()
        for rdma in ag_sends:
            rdma.wait_send()

        out_ref[0, :, :] = ag_ref[:].astype(jnp.float32)

    out = pl.pallas_call(
        body,
        out_shape=jax.ShapeDtypeStruct((1, SQ, D_MODEL), jnp.float32),
        in_specs=[pl.BlockSpec(memory_space=pltpu.VMEM)] * 5,
        out_specs=pl.BlockSpec(memory_space=pltpu.VMEM),
        scratch_shapes=[
            pltpu.VMEM((SQ, D_MODEL), jnp.bfloat16),
            pltpu.VMEM((SQ, D_MODEL), jnp.bfloat16),
            pltpu.VMEM((N_DEV - 1, CHUNK, D_MODEL), jnp.bfloat16),
            pltpu.SemaphoreType.DMA((N_DEV - 1,)),
            pltpu.SemaphoreType.DMA((N_DEV - 1,)),
            pltpu.SemaphoreType.DMA((N_DEV - 1,)),
            pltpu.SemaphoreType.DMA((N_DEV - 1,)),
        ],
        compiler_params=pltpu.CompilerParams(collective_id=0),
    )(xb, wqb, wob, wkb, wvb)
    return out
